# baseline (speedup 1.0000x reference)
import functools

import jax
import jax.numpy as jnp
from jax import lax
from jax.experimental import pallas as pl
from jax.experimental.pallas import tpu as pltpu
from jax.experimental.pallas import tpu_sc as plsc

V = 1000
DW = 16
OW = 2 * DW
N_TOTAL = 16384 * 200
TAB = 2 * V * DW

NC, NS = 2, 16
NW = NC * NS
PER_W = N_TOTAL // NW
CB = 512
NCHUNK = PER_W // CB

_mesh = plsc.VectorSubcoreMesh(core_axis_name="c", subcore_axis_name="s")


@functools.partial(
    pl.kernel,
    out_type=jax.ShapeDtypeStruct((N_TOTAL * 2 * OW,), jnp.bfloat16),
    mesh=_mesh,
    scratch_types=[
        pltpu.VMEM((TAB,), jnp.int32),
        pltpu.VMEM((2, CB), jnp.int32),
        pltpu.VMEM((2, CB * 2 * OW), jnp.bfloat16),
        pltpu.SemaphoreType.DMA,
        pltpu.SemaphoreType.DMA,
        pltpu.SemaphoreType.DMA,
        pltpu.SemaphoreType.DMA,
    ],
    compiler_params=pltpu.CompilerParams(
        needs_layout_passes=False, use_tc_tiling_on_sc=False
    ),
)
def _axial_kernel(idx_hbm, w_hbm, out_hbm, tab_v, idx_v, rows_v,
                  si0, si1, so0, so1):
    wid = lax.axis_index("s") * NC + lax.axis_index("c")
    base0 = wid * PER_W
    lane = lax.iota(jnp.int32, 16)
    si = (si0, si1)
    so = (so0, so1)

    dnums = lax.GatherDimensionNumbers(
        offset_dims=(), collapsed_slice_dims=(0,), start_index_map=(0,)
    )

    def bcast(vec, jj):
        pos = jnp.full((16, 1), jj, jnp.int32)
        return lax.gather(
            vec, pos, dnums, (1,),
            mode=lax.GatherScatterMode.PROMISE_IN_BOUNDS,
        )

    def idx_copy(ic, b):
        return pltpu.make_async_copy(
            idx_hbm.at[pl.ds(base0 + ic * CB, CB)], idx_v.at[b], si[b]
        )

    def out_copy(ic, b):
        return pltpu.make_async_copy(
            rows_v.at[b],
            out_hbm.at[pl.ds((base0 + ic * CB) * 2 * OW, CB * 2 * OW)],
            so[b],
        )

    pltpu.sync_copy(w_hbm, tab_v)
    idx_copy(0, 0).start()

    @pl.loop(0, NCHUNK, step=2)
    def _chunk(i):
        for b in (0, 1):
            ic = i + b

            @pl.when(ic >= 2)
            def _():
                out_copy(ic - 2, b).wait()

            idx_copy(ic, b).wait()

            @pl.when(ic + 1 < NCHUNK)
            def _():
                idx_copy(ic + 1, 1 - b).start()

            @plsc.parallel_loop(0, CB // 16)
            def _grp(j):
                v = idx_v[b, pl.ds(j * 16, 16)]
                lo = lax.rem(v, V) * DW
                hi = (lax.div(v, V) + V) * DW
                for jj in range(16):
                    a0 = bcast(lo, jj) + lane
                    b0 = bcast(hi, jj) + lane
                    off = 2 * (j * (16 * OW) + jj * OW)
                    rows_v[b, pl.ds(off, 32)] = plsc.bitcast(
                        plsc.load_gather(tab_v, [a0]), jnp.bfloat16)
                    rows_v[b, pl.ds(off + 32, 32)] = plsc.bitcast(
                        plsc.load_gather(tab_v, [b0]), jnp.bfloat16)

            out_copy(ic, b).start()

    out_copy(NCHUNK - 2, 0).wait()
    out_copy(NCHUNK - 1, 1).wait()


def kernel(idx, w0, w1):
    idx_flat = idx.reshape(-1).astype(jnp.int32)
    wb = jnp.concatenate([w0, w1], axis=0).astype(jnp.bfloat16)
    w_u32 = lax.bitcast_convert_type(
        wb.reshape(2 * V, DW, 2), jnp.int32
    ).reshape(-1)
    out_bf = _axial_kernel(idx_flat, w_u32)
    return out_bf.astype(jnp.float32).reshape(
        idx.shape[0], idx.shape[1], 2 * OW
    )


# speedup vs baseline: 1.4889x; 1.4889x over previous
import functools

import jax
import jax.numpy as jnp
from jax import lax
from jax.experimental import pallas as pl
from jax.experimental.pallas import tpu as pltpu
from jax.experimental.pallas import tpu_sc as plsc

V = 1000
D = 32
OD = 2 * D
N_TOTAL = 16384 * 200
TAB = 2 * V * D

NC, NS = 2, 16
NW = NC * NS
PER_W = N_TOTAL // NW
CB = 400
NCHUNK = PER_W // CB

_mesh = plsc.VectorSubcoreMesh(core_axis_name="c", subcore_axis_name="s")


@functools.partial(
    pl.kernel,
    out_type=jax.ShapeDtypeStruct((N_TOTAL * OD,), jnp.float32),
    mesh=_mesh,
    scratch_types=[
        pltpu.VMEM((TAB,), jnp.float32),
        pltpu.VMEM((2, CB), jnp.int32),
        pltpu.VMEM((2, CB * OD), jnp.float32),
        pltpu.SemaphoreType.DMA,
        pltpu.SemaphoreType.DMA,
        pltpu.SemaphoreType.DMA,
        pltpu.SemaphoreType.DMA,
    ],
    compiler_params=pltpu.CompilerParams(
        needs_layout_passes=False, use_tc_tiling_on_sc=False
    ),
)
def _axial_kernel(idx_hbm, w_hbm, out_hbm, tab_v, idx_v, rows_v,
                  si0, si1, so0, so1):
    wid = lax.axis_index("s") * NC + lax.axis_index("c")
    base0 = wid * PER_W
    lane = lax.iota(jnp.int32, 16)
    si = (si0, si1)
    so = (so0, so1)

    dnums = lax.GatherDimensionNumbers(
        offset_dims=(), collapsed_slice_dims=(0,), start_index_map=(0,)
    )

    def bcast(vec, jj):
        pos = jnp.full((16, 1), jj, jnp.int32)
        return lax.gather(
            vec, pos, dnums, (1,),
            mode=lax.GatherScatterMode.PROMISE_IN_BOUNDS,
        )

    def idx_copy(ic, b):
        return pltpu.make_async_copy(
            idx_hbm.at[pl.ds(base0 + ic * CB, CB)], idx_v.at[b], si[b]
        )

    def out_copy(ic, b):
        return pltpu.make_async_copy(
            rows_v.at[b],
            out_hbm.at[pl.ds((base0 + ic * CB) * OD, CB * OD)],
            so[b],
        )

    pltpu.sync_copy(w_hbm, tab_v)
    idx_copy(0, 0).start()

    @pl.loop(0, NCHUNK, step=2)
    def _chunk(i):
        for b in (0, 1):
            ic = i + b

            @pl.when(ic >= 2)
            def _():
                out_copy(ic - 2, b).wait()

            idx_copy(ic, b).wait()

            @pl.when(ic + 1 < NCHUNK)
            def _():
                idx_copy(ic + 1, 1 - b).start()

            @plsc.parallel_loop(0, CB // 16)
            def _grp(j):
                v = idx_v[b, pl.ds(j * 16, 16)]
                lo = lax.rem(v, V) * D
                hi = (lax.div(v, V) + V) * D
                for jj in range(16):
                    a0 = bcast(lo, jj) + lane
                    b0 = bcast(hi, jj) + lane
                    off = j * (16 * OD) + jj * OD
                    rows_v[b, pl.ds(off, 16)] = plsc.load_gather(tab_v, [a0])
                    rows_v[b, pl.ds(off + 16, 16)] = plsc.load_gather(
                        tab_v, [a0 + 16])
                    rows_v[b, pl.ds(off + 32, 16)] = plsc.load_gather(
                        tab_v, [b0])
                    rows_v[b, pl.ds(off + 48, 16)] = plsc.load_gather(
                        tab_v, [b0 + 16])

            out_copy(ic, b).start()

    out_copy(NCHUNK - 2, 0).wait()
    out_copy(NCHUNK - 1, 1).wait()


def kernel(idx, w0, w1):
    idx_flat = idx.reshape(-1).astype(jnp.int32)
    w = jnp.concatenate([w0, w1], axis=0).reshape(-1)
    out = _axial_kernel(idx_flat, w)
    return out.reshape(idx.shape[0], idx.shape[1], OD)
